# Initial kernel scaffold; baseline (speedup 1.0000x reference)
import jax
import jax.numpy as jnp
from jax.experimental import pallas as pl


def kernel(node_embedding, embedding_table_weight, batch, sequence, itemset_len, sequence_len, cue, W1_w, W1_b, W2_w, W2_b, q_w, q_b, W3_w, W3_b):
    raise NotImplementedError("write your pallas kernel here")


# speedup vs baseline: 3.7552x; 3.7552x over previous
import functools

import jax
import jax.numpy as jnp
from jax import lax
from jax.experimental import pallas as pl
from jax.experimental.pallas import tpu as pltpu

H = 128
B = 1024
N_PER = 32
L = 16
PAD = 8
SB = 8
TV = 2048
F32 = jnp.float32


def _dot_t(x, w):
    return lax.dot_general(x, w, (((1,), (1,)), ((), ())),
                           preferred_element_type=F32)


def _stage_a_body(seq_ref, il_ref, v_ref, w1_ref, w2_ref, b12_ref,
                  q_ref, qb_ref, w3a_ref, w3b_ref, b3_ref, sh_ref):
    niota = lax.broadcasted_iota(jnp.int32, (L, N_PER), 1)
    sess_list = []
    for s in range(SB):
        sq = seq_ref[s]
        counts = None
        for j in range(PAD):
            col = sq[:, j:j + 1]
            oh = jnp.where((col == niota) & (col < N_PER), 1.0, 0.0)
            counts = oh if counts is None else counts + oh
        a_s = counts / il_ref[s]
        v_s = v_ref[pl.ds(s * N_PER, N_PER), :]
        sess_list.append(
            lax.dot_general(a_s, v_s, (((1,), (0,)), ((), ())),
                            preferred_element_type=F32))
    sess = jnp.concatenate(sess_list, axis=0)
    v_n = jnp.concatenate([t[L - 1:L] for t in sess_list], axis=0)

    rows = lax.broadcasted_iota(jnp.int32, (SB * L, SB), 0)
    cols = lax.broadcasted_iota(jnp.int32, (SB * L, SB), 1)
    rep = jnp.where(rows // L == cols, 1.0, 0.0)
    v_n_rep = lax.dot_general(rep, v_n, (((1,), (0,)), ((), ())),
                              preferred_element_type=F32)

    a = jax.nn.sigmoid(_dot_t(v_n_rep, w1_ref[...]) + _dot_t(sess, w2_ref[...])
                       + b12_ref[...])
    alpha = jnp.sum(a * q_ref[...], axis=1, keepdims=True) + qb_ref[...]
    rows2 = lax.broadcasted_iota(jnp.int32, (SB, SB * L), 1)
    cols2 = lax.broadcasted_iota(jnp.int32, (SB, SB * L), 0)
    seg = jnp.where(rows2 // L == cols2, 1.0, 0.0)
    s_g = lax.dot_general(seg, alpha * sess, (((1,), (0,)), ((), ())),
                          preferred_element_type=F32)
    sh_ref[...] = (_dot_t(v_n, w3a_ref[...]) + _dot_t(s_g, w3b_ref[...])
                   + b3_ref[...])


def _stage_b_body(sh_ref, e_ref, cue_ref, out_ref, y_ref):
    k = pl.program_id(0)
    blk = _dot_t(sh_ref[...], e_ref[...])
    out_ref[...] = blk
    cols = lax.broadcasted_iota(jnp.int32, (B, TV), 1) + k * TV
    part = jnp.sum(jnp.where(cols == cue_ref[...], blk, 0.0),
                   axis=1, keepdims=True)

    @pl.when(k == 0)
    def _():
        y_ref[...] = part

    @pl.when(k > 0)
    def _():
        y_ref[...] = y_ref[...] + part


@jax.jit
def kernel(node_embedding, embedding_table_weight, batch, sequence, itemset_len,
           sequence_len, cue, W1_w, W1_b, W2_w, W2_b, q_w, q_b, W3_w, W3_b):
    del batch, sequence_len
    vocab = embedding_table_weight.shape[0]

    seq3 = sequence.reshape(B, L, PAD)
    il3 = itemset_len.reshape(B, L, 1).astype(F32)
    b12 = (W1_b + W2_b).reshape(1, H)
    qb2 = q_b.reshape(1, 1)
    w3a = W3_w[:, :H]
    w3b = W3_w[:, H:]
    b3 = W3_b.reshape(1, H)

    n_blocks = B // SB
    s_h = pl.pallas_call(
        _stage_a_body,
        grid=(n_blocks,),
        in_specs=[
            pl.BlockSpec((SB, L, PAD), lambda i: (i, 0, 0)),
            pl.BlockSpec((SB, L, 1), lambda i: (i, 0, 0)),
            pl.BlockSpec((SB * N_PER, H), lambda i: (i, 0)),
            pl.BlockSpec((H, H), lambda i: (0, 0)),
            pl.BlockSpec((H, H), lambda i: (0, 0)),
            pl.BlockSpec((1, H), lambda i: (0, 0)),
            pl.BlockSpec((1, H), lambda i: (0, 0)),
            pl.BlockSpec((1, 1), lambda i: (0, 0)),
            pl.BlockSpec((H, H), lambda i: (0, 0)),
            pl.BlockSpec((H, H), lambda i: (0, 0)),
            pl.BlockSpec((1, H), lambda i: (0, 0)),
        ],
        out_specs=pl.BlockSpec((SB, H), lambda i: (i, 0)),
        out_shape=jax.ShapeDtypeStruct((B, H), F32),
    )(seq3, il3, node_embedding, W1_w, W2_w, b12, q_w, qb2, w3a, w3b, b3)

    n_vtiles = pl.cdiv(vocab, TV)
    all_scores, y2 = pl.pallas_call(
        _stage_b_body,
        grid=(n_vtiles,),
        in_specs=[
            pl.BlockSpec((B, H), lambda k: (0, 0)),
            pl.BlockSpec((TV, H), lambda k: (k, 0)),
            pl.BlockSpec((B, 1), lambda k: (0, 0)),
        ],
        out_specs=[
            pl.BlockSpec((B, TV), lambda k: (0, k)),
            pl.BlockSpec((B, 1), lambda k: (0, 0)),
        ],
        out_shape=[
            jax.ShapeDtypeStruct((B, vocab), F32),
            jax.ShapeDtypeStruct((B, 1), F32),
        ],
    )(s_h, embedding_table_weight, cue.reshape(B, 1))

    return (y2.reshape(B), all_scores)
